# baseline (speedup 1.0000x reference)
import functools

import jax
import jax.numpy as jnp
from jax import lax
from jax.experimental import pallas as pl
from jax.experimental.pallas import tpu as pltpu
from jax.experimental.pallas import tpu_sc as plsc

D = 64
R = 16
NC = 2
NS = 16
NW = NC * NS
NBLK = 977
VPAD = NBLK * 1024

CT = 128
NSLOT = 2


def _eye(n):
    r = lax.broadcasted_iota(jnp.int32, (n, n), 0)
    c = lax.broadcasted_iota(jnp.int32, (n, n), 1)
    return jnp.where(r == c, 1.0, 0.0).astype(jnp.float32)


def _mxu_t(x):
    return lax.dot_general(
        x, _eye(x.shape[0]), (((0,), (0,)), ((), ())),
        preferred_element_type=jnp.float32)


def _tc_transpose_table(tT):
    def body(a_ref, b_ref, o_ref):
        o_ref[...] = jnp.concatenate(
            [_mxu_t(a_ref[...]), _mxu_t(b_ref[...])], axis=1)

    return pl.pallas_call(
        body,
        grid=(NBLK,),
        in_specs=[
            pl.BlockSpec((D, 512), lambda j: (0, 2 * j)),
            pl.BlockSpec((D, 512), lambda j: (0, 2 * j + 1)),
        ],
        out_specs=pl.BlockSpec((512, 128), lambda j: (j, 0)),
        out_shape=jax.ShapeDtypeStruct((VPAD // 2, 128), jnp.float32),
    )(tT, tT)


def _tc_transpose_lora(aT):
    def body(*refs):
        o_ref = refs[8]
        o_ref[...] = jnp.concatenate(
            [_mxu_t(r[...]) for r in refs[:8]], axis=1)

    return pl.pallas_call(
        body,
        grid=(NBLK,),
        in_specs=[
            pl.BlockSpec((R, 128), lambda j, k=k: (0, 8 * j + k))
            for k in range(8)
        ],
        out_specs=pl.BlockSpec((128, 128), lambda j: (j, 0)),
        out_shape=jax.ShapeDtypeStruct((VPAD // 8, 128), jnp.float32),
    )(*([aT] * 8))


@functools.partial(jax.jit, static_argnums=(3, 4))
def _sc_gather(idx3, t64, a16, n_tokens, n_rows):
    per_w = n_rows * 128
    n_chunks = per_w // CT
    mesh = plsc.VectorSubcoreMesh(core_axis_name="c", subcore_axis_name="s")

    @functools.partial(
        pl.kernel,
        mesh=mesh,
        out_type=(
            jax.ShapeDtypeStruct((n_tokens, D), jnp.float32),
            jax.ShapeDtypeStruct((n_tokens // 2, 128), jnp.float32),
        ),
        scratch_types=[
            pltpu.VMEM((n_rows, 128), jnp.int32),
            pltpu.VMEM((n_rows, 128), jnp.int32),
            pltpu.VMEM((n_rows, 128), jnp.int32),
            pltpu.VMEM((NSLOT, CT, D), jnp.float32),
            pltpu.VMEM((NSLOT, CT, R), jnp.float32),
            pltpu.VMEM((NSLOT, CT // 2, 128), jnp.float32),
            pltpu.SemaphoreType.DMA,
            pltpu.SemaphoreType.DMA,
            pltpu.SemaphoreType.DMA,
            pltpu.SemaphoreType.DMA,
        ],
        compiler_params=pltpu.CompilerParams(use_tc_tiling_on_sc=False),
    )
    def k(idx_hbm, t_hbm, a_hbm, base_out, hid_out, idx_v, mt_v, ml_v,
          rows_v, hv_v, pv_v, g0, g1, w0, w1):
        gsem = (g0, g1)
        wsem = (w0, w1)
        cid = lax.axis_index("c")
        sid = lax.axis_index("s")
        wid = sid * NC + cid
        pltpu.sync_copy(idx_hbm.at[wid], idx_v)

        def xform(r, _):
            for g in range(8):
                v = idx_v[r, pl.ds(g * 16, 16)]
                hi = (v >> 10) << 10
                mt = hi + ((v & 511) << 1) + ((v >> 9) & 1)
                ml = hi + ((v & 127) << 3) + ((v >> 7) & 7)
                mt_v[r, pl.ds(g * 16, 16)] = mt
                ml_v[r, pl.ds(g * 16, 16)] = ml
            return 0

        lax.fori_loop(0, n_rows, xform, 0)

        zero = jnp.zeros((16,), jnp.float32)

        def zpad(r, _):
            for s in range(NSLOT):
                for g in range(2, 8):
                    pv_v[s, r, pl.ds(g * 16, 16)] = zero
            return 0

        lax.fori_loop(0, CT // 2, zpad, 0)

        def fire_gather(kk, s):
            pltpu.make_async_copy(
                t_hbm.at[mt_v.at[kk]], rows_v.at[s], gsem[s]).start()
            pltpu.make_async_copy(
                a_hbm.at[ml_v.at[kk]], hv_v.at[s], gsem[s]).start()

        def wait_gather(kk, s):
            pltpu.make_async_copy(
                t_hbm.at[mt_v.at[kk]], rows_v.at[s], gsem[s]).wait()
            pltpu.make_async_copy(
                a_hbm.at[ml_v.at[kk]], hv_v.at[s], gsem[s]).wait()

        def pack_pairs(s):
            for r in range(CT // 2):
                pv_v[s, r, pl.ds(0, 16)] = hv_v[s, 2 * r, :]
                pv_v[s, r, pl.ds(16, 16)] = hv_v[s, 2 * r + 1, :]

        def _base_slice(kk):
            return pl.ds(pl.multiple_of(wid * per_w + kk * CT, CT), CT)

        def _hid_slice(kk):
            return pl.ds(
                pl.multiple_of((wid * per_w + kk * CT) // 2, CT // 2), CT // 2)

        def fire_write(kk, s):
            pltpu.make_async_copy(
                rows_v.at[s], base_out.at[_base_slice(kk)], wsem[s]).start()
            pltpu.make_async_copy(
                pv_v.at[s], hid_out.at[_hid_slice(kk)], wsem[s]).start()

        def wait_write(kk, s):
            pltpu.make_async_copy(
                rows_v.at[s], base_out.at[_base_slice(kk)], wsem[s]).wait()
            pltpu.make_async_copy(
                pv_v.at[s], hid_out.at[_hid_slice(kk)], wsem[s]).wait()

        def step(kk, s, fire_next):
            wait_gather(kk, s)
            pack_pairs(s)
            fire_write(kk, s)
            wait_write(kk, s)
            if fire_next:
                fire_gather(kk + NSLOT, s)

        for s in range(NSLOT):
            fire_gather(s, s)

        def chunk(i, _):
            for s in range(NSLOT):
                step(NSLOT * i + s, s, True)
            return 0

        lax.fori_loop(0, n_chunks // NSLOT - 1, chunk, 0)
        for s in range(NSLOT):
            step(n_chunks - NSLOT + s, s, False)

    return k(idx3, t64, a16)


def _build_w(lora_b):
    w = jnp.zeros((128, 128), jnp.float32)
    w = w.at[0:R, 0:D].set(lora_b)
    w = w.at[R:2 * R, D:2 * D].set(lora_b)
    return w


def _tc_combine(base2, hid2, w, n2):
    blk = 1024

    def body(b_ref, h_ref, w_ref, o_ref):
        o_ref[...] = b_ref[...] + jnp.dot(
            h_ref[...], w_ref[...], preferred_element_type=jnp.float32)

    return pl.pallas_call(
        body,
        grid=(n2 // blk,),
        in_specs=[
            pl.BlockSpec((blk, 128), lambda i: (i, 0)),
            pl.BlockSpec((blk, 128), lambda i: (i, 0)),
            pl.BlockSpec((128, 128), lambda i: (0, 0)),
        ],
        out_specs=pl.BlockSpec((blk, 128), lambda i: (i, 0)),
        out_shape=jax.ShapeDtypeStruct((n2, 128), jnp.float32),
    )(base2, hid2, w)


def kernel(x, table, lora_A, lora_B):
    b, s = x.shape
    n = b * s
    n_rows = n // (NW * 128)
    tconv = _tc_transpose_table(table.T)
    aconv = _tc_transpose_lora(lora_A.T)
    t64 = tconv.reshape(VPAD, D)
    a16 = aconv.reshape(VPAD, R)
    idx3 = x.reshape(NW, n_rows, 128).astype(jnp.int32)
    base, hid2 = _sc_gather(idx3, t64, a16, n, n_rows)
    base2 = base.reshape(n // 2, 128)
    out2 = _tc_combine(base2, hid2, _build_w(lora_B), n // 2)
    return out2.reshape(b, s, D)


# speedup vs baseline: 1.9931x; 1.9931x over previous
import functools

import jax
import jax.numpy as jnp
from jax import lax
from jax.experimental import pallas as pl
from jax.experimental.pallas import tpu as pltpu
from jax.experimental.pallas import tpu_sc as plsc

D = 64
R = 16
NC = 2
NS = 16
NW = NC * NS
BLKW = 8192
NBLK = 123
VPAD = NBLK * BLKW

CT = 128
NSLOT = 2


def _eye(n):
    r = lax.broadcasted_iota(jnp.int32, (n, n), 0)
    c = lax.broadcasted_iota(jnp.int32, (n, n), 1)
    return jnp.where(r == c, 1.0, 0.0).astype(jnp.float32)


def _mxu_t(x):
    return lax.dot_general(
        x, _eye(x.shape[0]), (((0,), (0,)), ((), ())),
        preferred_element_type=jnp.float32)


def _tc_transpose_table(tT):
    def body(a_ref, b_ref, o_ref):
        o_ref[...] = jnp.concatenate(
            [_mxu_t(a_ref[...]), _mxu_t(b_ref[...])], axis=1)

    return pl.pallas_call(
        body,
        grid=(NBLK,),
        in_specs=[
            pl.BlockSpec((D, BLKW // 2), lambda j: (0, jnp.minimum(2 * j, 244))),
            pl.BlockSpec(
                (D, BLKW // 2), lambda j: (0, jnp.minimum(2 * j + 1, 244))),
        ],
        out_specs=pl.BlockSpec((BLKW // 2, 128), lambda j: (j, 0)),
        out_shape=jax.ShapeDtypeStruct((VPAD // 2, 128), jnp.float32),
    )(tT, tT)


def _tc_transpose_lora(aT):
    def body(*refs):
        o_ref = refs[8]
        o_ref[...] = jnp.concatenate(
            [_mxu_t(r[...]) for r in refs[:8]], axis=1)

    return pl.pallas_call(
        body,
        grid=(NBLK,),
        in_specs=[
            pl.BlockSpec(
                (R, BLKW // 8), lambda j, k=k: (0, jnp.minimum(8 * j + k, 976)))
            for k in range(8)
        ],
        out_specs=pl.BlockSpec((BLKW // 8, 128), lambda j: (j, 0)),
        out_shape=jax.ShapeDtypeStruct((VPAD // 8, 128), jnp.float32),
    )(*([aT] * 8))


@functools.partial(jax.jit, static_argnums=(3, 4))
def _sc_gather(idx3, t64, a16, n_tokens, n_rows):
    per_w = n_rows * 128
    n_chunks = per_w // CT
    mesh = plsc.VectorSubcoreMesh(core_axis_name="c", subcore_axis_name="s")

    @functools.partial(
        pl.kernel,
        mesh=mesh,
        out_type=(
            jax.ShapeDtypeStruct((n_tokens, D), jnp.float32),
            jax.ShapeDtypeStruct((n_tokens // 2, 128), jnp.float32),
        ),
        scratch_types=[
            pltpu.VMEM((n_rows, 128), jnp.int32),
            pltpu.VMEM((n_rows, 128), jnp.int32),
            pltpu.VMEM((n_rows, 128), jnp.int32),
            pltpu.VMEM((NSLOT, CT, D), jnp.float32),
            pltpu.VMEM((NSLOT, CT, R), jnp.float32),
            pltpu.VMEM((NSLOT, CT // 2, 128), jnp.float32),
            pltpu.SemaphoreType.DMA,
            pltpu.SemaphoreType.DMA,
            pltpu.SemaphoreType.DMA,
            pltpu.SemaphoreType.DMA,
        ],
        compiler_params=pltpu.CompilerParams(use_tc_tiling_on_sc=False),
    )
    def k(idx_hbm, t_hbm, a_hbm, base_out, hid_out, idx_v, mt_v, ml_v,
          rows_v, hv_v, pv_v, g0, g1, w0, w1):
        gsem = (g0, g1)
        wsem = (w0, w1)
        cid = lax.axis_index("c")
        sid = lax.axis_index("s")
        wid = sid * NC + cid
        pltpu.sync_copy(idx_hbm.at[wid], idx_v)

        def xform(r, _):
            for g in range(8):
                v = idx_v[r, pl.ds(g * 16, 16)]
                hi = (v >> 13) << 13
                mt = hi + ((v & 4095) << 1) + ((v >> 12) & 1)
                ml = hi + ((v & 1023) << 3) + ((v >> 10) & 7)
                mt_v[r, pl.ds(g * 16, 16)] = mt
                ml_v[r, pl.ds(g * 16, 16)] = ml
            return 0

        lax.fori_loop(0, n_rows, xform, 0)

        zero = jnp.zeros((16,), jnp.float32)

        def zpad(r, _):
            for s in range(NSLOT):
                for g in range(2, 8):
                    pv_v[s, r, pl.ds(g * 16, 16)] = zero
            return 0

        lax.fori_loop(0, CT // 2, zpad, 0)

        def fire_gather(kk, s):
            pltpu.make_async_copy(
                t_hbm.at[mt_v.at[kk]], rows_v.at[s], gsem[s]).start()
            pltpu.make_async_copy(
                a_hbm.at[ml_v.at[kk]], hv_v.at[s], gsem[s]).start()

        def wait_gather(kk, s):
            pltpu.make_async_copy(
                t_hbm.at[mt_v.at[kk]], rows_v.at[s], gsem[s]).wait()
            pltpu.make_async_copy(
                a_hbm.at[ml_v.at[kk]], hv_v.at[s], gsem[s]).wait()

        def pack_pairs(s):
            for r in range(CT // 2):
                pv_v[s, r, pl.ds(0, 16)] = hv_v[s, 2 * r, :]
                pv_v[s, r, pl.ds(16, 16)] = hv_v[s, 2 * r + 1, :]

        def _base_slice(kk):
            return pl.ds(pl.multiple_of(wid * per_w + kk * CT, CT), CT)

        def _hid_slice(kk):
            return pl.ds(
                pl.multiple_of((wid * per_w + kk * CT) // 2, CT // 2), CT // 2)

        def fire_write(kk, s):
            pltpu.make_async_copy(
                rows_v.at[s], base_out.at[_base_slice(kk)], wsem[s]).start()
            pltpu.make_async_copy(
                pv_v.at[s], hid_out.at[_hid_slice(kk)], wsem[s]).start()

        def wait_write(kk, s):
            pltpu.make_async_copy(
                rows_v.at[s], base_out.at[_base_slice(kk)], wsem[s]).wait()
            pltpu.make_async_copy(
                pv_v.at[s], hid_out.at[_hid_slice(kk)], wsem[s]).wait()

        def step(kk, s, fire_next):
            wait_gather(kk, s)
            pack_pairs(s)
            fire_write(kk, s)
            wait_write(kk, s)
            if fire_next:
                fire_gather(kk + NSLOT, s)

        for s in range(NSLOT):
            fire_gather(s, s)

        def chunk(i, _):
            for s in range(NSLOT):
                step(NSLOT * i + s, s, True)
            return 0

        lax.fori_loop(0, n_chunks // NSLOT - 1, chunk, 0)
        for s in range(NSLOT):
            step(n_chunks - NSLOT + s, s, False)

    return k(idx3, t64, a16)


def _build_w(lora_b):
    w = jnp.zeros((128, 128), jnp.float32)
    w = w.at[0:R, 0:D].set(lora_b)
    w = w.at[R:2 * R, D:2 * D].set(lora_b)
    return w


def _tc_combine(base2, hid2, w, n2):
    blk = 1024

    def body(b_ref, h_ref, w_ref, o_ref):
        o_ref[...] = b_ref[...] + jnp.dot(
            h_ref[...], w_ref[...], preferred_element_type=jnp.float32)

    return pl.pallas_call(
        body,
        grid=(n2 // blk,),
        in_specs=[
            pl.BlockSpec((blk, 128), lambda i: (i, 0)),
            pl.BlockSpec((blk, 128), lambda i: (i, 0)),
            pl.BlockSpec((128, 128), lambda i: (0, 0)),
        ],
        out_specs=pl.BlockSpec((blk, 128), lambda i: (i, 0)),
        out_shape=jax.ShapeDtypeStruct((n2, 128), jnp.float32),
    )(base2, hid2, w)


def kernel(x, table, lora_A, lora_B):
    b, s = x.shape
    n = b * s
    n_rows = n // (NW * 128)
    tconv = _tc_transpose_table(table.T)
    aconv = _tc_transpose_lora(lora_A.T)
    t64 = tconv.reshape(VPAD, D)
    a16 = aconv.reshape(VPAD, R)
    idx3 = x.reshape(NW, n_rows, 128).astype(jnp.int32)
    base, hid2 = _sc_gather(idx3, t64, a16, n, n_rows)
    base2 = base.reshape(n // 2, 128)
    out2 = _tc_combine(base2, hid2, _build_w(lora_B), n // 2)
    return out2.reshape(b, s, D)


# speedup vs baseline: 2.0473x; 1.0272x over previous
import functools

import jax
import jax.numpy as jnp
from jax import lax
from jax.experimental import pallas as pl
from jax.experimental.pallas import tpu as pltpu
from jax.experimental.pallas import tpu_sc as plsc

D = 64
R = 16
NC = 2
NS = 16
NW = NC * NS
BLKW = 8192
NBLK = 123
VPAD = NBLK * BLKW

CT = 128
NSLOT = 2


def _eye(n):
    r = lax.broadcasted_iota(jnp.int32, (n, n), 0)
    c = lax.broadcasted_iota(jnp.int32, (n, n), 1)
    return jnp.where(r == c, 1.0, 0.0).astype(jnp.float32)


def _mxu_t(x):
    return lax.dot_general(
        x, _eye(x.shape[0]), (((0,), (0,)), ((), ())),
        preferred_element_type=jnp.float32)


def _tc_transpose_table(tT):
    def body(a_ref, b_ref, o_ref):
        o_ref[...] = jnp.concatenate(
            [_mxu_t(a_ref[...]), _mxu_t(b_ref[...])], axis=1)

    return pl.pallas_call(
        body,
        grid=(NBLK,),
        in_specs=[
            pl.BlockSpec((D, BLKW // 2), lambda j: (0, jnp.minimum(2 * j, 244))),
            pl.BlockSpec(
                (D, BLKW // 2), lambda j: (0, jnp.minimum(2 * j + 1, 244))),
        ],
        out_specs=pl.BlockSpec((BLKW // 2, 128), lambda j: (j, 0)),
        out_shape=jax.ShapeDtypeStruct((VPAD // 2, 128), jnp.float32),
    )(tT, tT)


def _tc_transpose_lora(aT):
    def body(x_ref, o_ref):
        big = _mxu_t(x_ref[...])
        w = BLKW // 8
        o_ref[...] = jnp.concatenate(
            [lax.slice(big, (k * w, 0), ((k + 1) * w, R)) for k in range(8)],
            axis=1)

    return pl.pallas_call(
        body,
        grid=(NBLK,),
        in_specs=[pl.BlockSpec((R, BLKW), lambda j: (0, j))],
        out_specs=pl.BlockSpec((BLKW // 8, 128), lambda j: (j, 0)),
        out_shape=jax.ShapeDtypeStruct((VPAD // 8, 128), jnp.float32),
    )(aT)


@functools.partial(jax.jit, static_argnums=(3, 4))
def _sc_gather(idx3, t64, a16, n_tokens, n_rows):
    per_w = n_rows * 128
    n_chunks = per_w // CT
    mesh = plsc.VectorSubcoreMesh(core_axis_name="c", subcore_axis_name="s")

    @functools.partial(
        pl.kernel,
        mesh=mesh,
        out_type=(
            jax.ShapeDtypeStruct((n_tokens, D), jnp.float32),
            jax.ShapeDtypeStruct((n_tokens // 2, 128), jnp.float32),
        ),
        scratch_types=[
            pltpu.VMEM((n_rows, 128), jnp.int32),
            pltpu.VMEM((n_rows, 128), jnp.int32),
            pltpu.VMEM((n_rows, 128), jnp.int32),
            pltpu.VMEM((NSLOT, CT, D), jnp.float32),
            pltpu.VMEM((NSLOT, CT, R), jnp.float32),
            pltpu.VMEM((NSLOT, CT // 2, 128), jnp.float32),
            pltpu.SemaphoreType.DMA,
            pltpu.SemaphoreType.DMA,
            pltpu.SemaphoreType.DMA,
            pltpu.SemaphoreType.DMA,
        ],
        compiler_params=pltpu.CompilerParams(use_tc_tiling_on_sc=False),
    )
    def k(idx_hbm, t_hbm, a_hbm, base_out, hid_out, idx_v, mt_v, ml_v,
          rows_v, hv_v, pv_v, g0, g1, w0, w1):
        gsem = (g0, g1)
        wsem = (w0, w1)
        cid = lax.axis_index("c")
        sid = lax.axis_index("s")
        wid = sid * NC + cid
        pltpu.sync_copy(idx_hbm.at[wid], idx_v)

        def xform(r, _):
            for g in range(8):
                v = idx_v[r, pl.ds(g * 16, 16)]
                hi = (v >> 13) << 13
                mt = hi + ((v & 4095) << 1) + ((v >> 12) & 1)
                ml = hi + ((v & 1023) << 3) + ((v >> 10) & 7)
                mt_v[r, pl.ds(g * 16, 16)] = mt
                ml_v[r, pl.ds(g * 16, 16)] = ml
            return 0

        lax.fori_loop(0, n_rows, xform, 0)

        zero = jnp.zeros((16,), jnp.float32)

        def zpad(r, _):
            for s in range(NSLOT):
                for g in range(2, 8):
                    pv_v[s, r, pl.ds(g * 16, 16)] = zero
            return 0

        lax.fori_loop(0, CT // 2, zpad, 0)

        def fire_gather(kk, s):
            pltpu.make_async_copy(
                t_hbm.at[mt_v.at[kk]], rows_v.at[s], gsem[s]).start()
            pltpu.make_async_copy(
                a_hbm.at[ml_v.at[kk]], hv_v.at[s], gsem[s]).start()

        def wait_gather(kk, s):
            pltpu.make_async_copy(
                t_hbm.at[mt_v.at[kk]], rows_v.at[s], gsem[s]).wait()
            pltpu.make_async_copy(
                a_hbm.at[ml_v.at[kk]], hv_v.at[s], gsem[s]).wait()

        def pack_pairs(s):
            for r in range(CT // 2):
                pv_v[s, r, pl.ds(0, 16)] = hv_v[s, 2 * r, :]
                pv_v[s, r, pl.ds(16, 16)] = hv_v[s, 2 * r + 1, :]

        def _base_slice(kk):
            return pl.ds(pl.multiple_of(wid * per_w + kk * CT, CT), CT)

        def _hid_slice(kk):
            return pl.ds(
                pl.multiple_of((wid * per_w + kk * CT) // 2, CT // 2), CT // 2)

        def fire_write(kk, s):
            pltpu.make_async_copy(
                rows_v.at[s], base_out.at[_base_slice(kk)], wsem[s]).start()
            pltpu.make_async_copy(
                pv_v.at[s], hid_out.at[_hid_slice(kk)], wsem[s]).start()

        def wait_write(kk, s):
            pltpu.make_async_copy(
                rows_v.at[s], base_out.at[_base_slice(kk)], wsem[s]).wait()
            pltpu.make_async_copy(
                pv_v.at[s], hid_out.at[_hid_slice(kk)], wsem[s]).wait()

        def step(kk, s, fire_next):
            wait_gather(kk, s)
            pack_pairs(s)
            fire_write(kk, s)
            wait_write(kk, s)
            if fire_next:
                fire_gather(kk + NSLOT, s)

        for s in range(NSLOT):
            fire_gather(s, s)

        def chunk(i, _):
            for s in range(NSLOT):
                step(NSLOT * i + s, s, True)
            return 0

        lax.fori_loop(0, n_chunks // NSLOT - 1, chunk, 0)
        for s in range(NSLOT):
            step(n_chunks - NSLOT + s, s, False)

    return k(idx3, t64, a16)


def _build_w(lora_b):
    w = jnp.zeros((128, 128), jnp.float32)
    w = w.at[0:R, 0:D].set(lora_b)
    w = w.at[R:2 * R, D:2 * D].set(lora_b)
    return w


def _tc_combine(base2, hid2, w, n2):
    blk = 1024

    def body(b_ref, h_ref, w_ref, o_ref):
        o_ref[...] = b_ref[...] + jnp.dot(
            h_ref[...], w_ref[...], preferred_element_type=jnp.float32)

    return pl.pallas_call(
        body,
        grid=(n2 // blk,),
        in_specs=[
            pl.BlockSpec((blk, 128), lambda i: (i, 0)),
            pl.BlockSpec((blk, 128), lambda i: (i, 0)),
            pl.BlockSpec((128, 128), lambda i: (0, 0)),
        ],
        out_specs=pl.BlockSpec((blk, 128), lambda i: (i, 0)),
        out_shape=jax.ShapeDtypeStruct((n2, 128), jnp.float32),
    )(base2, hid2, w)


def kernel(x, table, lora_A, lora_B):
    b, s = x.shape
    n = b * s
    n_rows = n // (NW * 128)
    tconv = _tc_transpose_table(table.T)
    aconv = _tc_transpose_lora(lora_A.T)
    t64 = tconv.reshape(VPAD, D)
    a16 = aconv.reshape(VPAD, R)
    idx3 = x.reshape(NW, n_rows, 128).astype(jnp.int32)
    base, hid2 = _sc_gather(idx3, t64, a16, n, n_rows)
    base2 = base.reshape(n // 2, 128)
    out2 = _tc_combine(base2, hid2, _build_w(lora_B), n // 2)
    return out2.reshape(b, s, D)


# speedup vs baseline: 2.8354x; 1.3849x over previous
import functools

import jax
import jax.numpy as jnp
from jax import lax
from jax.experimental import pallas as pl
from jax.experimental.pallas import tpu as pltpu
from jax.experimental.pallas import tpu_sc as plsc

D = 64
R = 16
NC = 2
NS = 16
NW = NC * NS
BLKW = 8192
NBLK = 123
VPAD = NBLK * BLKW

CT = 128
NSLOT = 2


def _eye(n):
    r = lax.broadcasted_iota(jnp.int32, (n, n), 0)
    c = lax.broadcasted_iota(jnp.int32, (n, n), 1)
    return jnp.where(r == c, 1.0, 0.0).astype(jnp.float32)


def _mxu_t(x):
    return lax.dot_general(
        x, _eye(x.shape[0]), (((0,), (0,)), ((), ())),
        preferred_element_type=jnp.float32)


def _tc_transpose_table(tT):
    def body(a_ref, b_ref, o_ref):
        stacked = jnp.concatenate([a_ref[...], b_ref[...]], axis=0)
        o_ref[...] = _mxu_t(stacked)

    return pl.pallas_call(
        body,
        grid=(NBLK,),
        in_specs=[
            pl.BlockSpec((D, BLKW // 2), lambda j: (0, jnp.minimum(2 * j, 244))),
            pl.BlockSpec(
                (D, BLKW // 2), lambda j: (0, jnp.minimum(2 * j + 1, 244))),
        ],
        out_specs=pl.BlockSpec((BLKW // 2, 128), lambda j: (j, 0)),
        out_shape=jax.ShapeDtypeStruct((VPAD // 2, 128), jnp.float32),
    )(tT, tT)


def _tc_transpose_lora(aT):
    def body(x_ref, o_ref):
        x = x_ref[...]
        w = BLKW // 8
        stacked = jnp.concatenate(
            [lax.slice(x, (0, k * w), (R, (k + 1) * w)) for k in range(8)],
            axis=0)
        o_ref[...] = _mxu_t(stacked)

    return pl.pallas_call(
        body,
        grid=(NBLK,),
        in_specs=[pl.BlockSpec((R, BLKW), lambda j: (0, j))],
        out_specs=pl.BlockSpec((BLKW // 8, 128), lambda j: (j, 0)),
        out_shape=jax.ShapeDtypeStruct((VPAD // 8, 128), jnp.float32),
    )(aT)


@functools.partial(jax.jit, static_argnums=(3, 4))
def _sc_gather(idx3, t64, a16, n_tokens, n_rows):
    per_w = n_rows * 128
    n_chunks = per_w // CT
    mesh = plsc.VectorSubcoreMesh(core_axis_name="c", subcore_axis_name="s")

    @functools.partial(
        pl.kernel,
        mesh=mesh,
        out_type=(
            jax.ShapeDtypeStruct((n_tokens, D), jnp.float32),
            jax.ShapeDtypeStruct((n_tokens // 2, 128), jnp.float32),
        ),
        scratch_types=[
            pltpu.VMEM((n_rows, 128), jnp.int32),
            pltpu.VMEM((n_rows, 128), jnp.int32),
            pltpu.VMEM((n_rows, 128), jnp.int32),
            pltpu.VMEM((NSLOT, CT, D), jnp.float32),
            pltpu.VMEM((NSLOT, CT, R), jnp.float32),
            pltpu.VMEM((NSLOT, CT // 2, 128), jnp.float32),
            pltpu.SemaphoreType.DMA,
            pltpu.SemaphoreType.DMA,
            pltpu.SemaphoreType.DMA,
            pltpu.SemaphoreType.DMA,
        ],
        compiler_params=pltpu.CompilerParams(use_tc_tiling_on_sc=False),
    )
    def k(idx_hbm, t_hbm, a_hbm, base_out, hid_out, idx_v, mt_v, ml_v,
          rows_v, hv_v, pv_v, g0, g1, w0, w1):
        gsem = (g0, g1)
        wsem = (w0, w1)
        cid = lax.axis_index("c")
        sid = lax.axis_index("s")
        wid = sid * NC + cid
        pltpu.sync_copy(idx_hbm.at[wid], idx_v)

        def xform(r, _):
            for g in range(8):
                v = idx_v[r, pl.ds(g * 16, 16)]
                hi = (v >> 13) << 13
                mt = hi + ((v & 4095) << 1) + ((v >> 12) & 1)
                ml = hi + ((v & 1023) << 3) + ((v >> 10) & 7)
                mt_v[r, pl.ds(g * 16, 16)] = mt
                ml_v[r, pl.ds(g * 16, 16)] = ml
            return 0

        lax.fori_loop(0, n_rows, xform, 0)

        zero = jnp.zeros((16,), jnp.float32)

        def zpad(r, _):
            for s in range(NSLOT):
                for g in range(2, 8):
                    pv_v[s, r, pl.ds(g * 16, 16)] = zero
            return 0

        lax.fori_loop(0, CT // 2, zpad, 0)

        def fire_gather(kk, s):
            pltpu.make_async_copy(
                t_hbm.at[mt_v.at[kk]], rows_v.at[s], gsem[s]).start()
            pltpu.make_async_copy(
                a_hbm.at[ml_v.at[kk]], hv_v.at[s], gsem[s]).start()

        def wait_gather(kk, s):
            pltpu.make_async_copy(
                t_hbm.at[mt_v.at[kk]], rows_v.at[s], gsem[s]).wait()
            pltpu.make_async_copy(
                a_hbm.at[ml_v.at[kk]], hv_v.at[s], gsem[s]).wait()

        def pack_pairs(s):
            for r in range(CT // 2):
                pv_v[s, r, pl.ds(0, 16)] = hv_v[s, 2 * r, :]
                pv_v[s, r, pl.ds(16, 16)] = hv_v[s, 2 * r + 1, :]

        def _base_slice(kk):
            return pl.ds(pl.multiple_of(wid * per_w + kk * CT, CT), CT)

        def _hid_slice(kk):
            return pl.ds(
                pl.multiple_of((wid * per_w + kk * CT) // 2, CT // 2), CT // 2)

        def fire_write(kk, s):
            pltpu.make_async_copy(
                rows_v.at[s], base_out.at[_base_slice(kk)], wsem[s]).start()
            pltpu.make_async_copy(
                pv_v.at[s], hid_out.at[_hid_slice(kk)], wsem[s]).start()

        def wait_write(kk, s):
            pltpu.make_async_copy(
                rows_v.at[s], base_out.at[_base_slice(kk)], wsem[s]).wait()
            pltpu.make_async_copy(
                pv_v.at[s], hid_out.at[_hid_slice(kk)], wsem[s]).wait()

        def step(kk, s, fire_next):
            wait_gather(kk, s)
            pack_pairs(s)
            fire_write(kk, s)
            wait_write(kk, s)
            if fire_next:
                fire_gather(kk + NSLOT, s)

        for s in range(NSLOT):
            fire_gather(s, s)

        def chunk(i, _):
            for s in range(NSLOT):
                step(NSLOT * i + s, s, True)
            return 0

        lax.fori_loop(0, n_chunks // NSLOT - 1, chunk, 0)
        for s in range(NSLOT):
            step(n_chunks - NSLOT + s, s, False)

    return k(idx3, t64, a16)


def _build_w(lora_b):
    w = jnp.zeros((128, 128), jnp.float32)
    w = w.at[0:R, 0:D].set(lora_b)
    w = w.at[R:2 * R, D:2 * D].set(lora_b)
    return w


def _tc_combine(base2, hid2, w, n2):
    blk = 1024

    def body(b_ref, h_ref, w_ref, o_ref):
        o_ref[...] = b_ref[...] + jnp.dot(
            h_ref[...], w_ref[...], preferred_element_type=jnp.float32)

    return pl.pallas_call(
        body,
        grid=(n2 // blk,),
        in_specs=[
            pl.BlockSpec((blk, 128), lambda i: (i, 0)),
            pl.BlockSpec((blk, 128), lambda i: (i, 0)),
            pl.BlockSpec((128, 128), lambda i: (0, 0)),
        ],
        out_specs=pl.BlockSpec((blk, 128), lambda i: (i, 0)),
        out_shape=jax.ShapeDtypeStruct((n2, 128), jnp.float32),
    )(base2, hid2, w)


def kernel(x, table, lora_A, lora_B):
    b, s = x.shape
    n = b * s
    n_rows = n // (NW * 128)
    tconv = _tc_transpose_table(table.T)
    aconv = _tc_transpose_lora(lora_A.T)
    t64 = tconv.reshape(VPAD, D)
    a16 = aconv.reshape(VPAD, R)
    idx3 = x.reshape(NW, n_rows, 128).astype(jnp.int32)
    base, hid2 = _sc_gather(idx3, t64, a16, n, n_rows)
    base2 = base.reshape(n // 2, 128)
    out2 = _tc_combine(base2, hid2, _build_w(lora_B), n // 2)
    return out2.reshape(b, s, D)


# speedup vs baseline: 3.3981x; 1.1984x over previous
import functools

import jax
import jax.numpy as jnp
from jax import lax
from jax.experimental import pallas as pl
from jax.experimental.pallas import tpu as pltpu
from jax.experimental.pallas import tpu_sc as plsc

D = 64
R = 16
NC = 2
NS = 16
NW = NC * NS
BLKW = 16384
NBLK = 62
VPAD = NBLK * BLKW

CT = 128
NSLOT = 2


def _eye(n):
    r = lax.broadcasted_iota(jnp.int32, (n, n), 0)
    c = lax.broadcasted_iota(jnp.int32, (n, n), 1)
    return jnp.where(r == c, 1.0, 0.0).astype(jnp.float32)


def _mxu_t(x):
    return lax.dot_general(
        x, _eye(x.shape[0]), (((0,), (0,)), ((), ())),
        preferred_element_type=jnp.float32)


def _tc_transpose_table(tT):
    def body(a_ref, b_ref, o_ref):
        stacked = jnp.concatenate([a_ref[...], b_ref[...]], axis=0)
        o_ref[...] = _mxu_t(stacked)

    return pl.pallas_call(
        body,
        grid=(NBLK,),
        in_specs=[
            pl.BlockSpec((D, BLKW // 2), lambda j: (0, jnp.minimum(2 * j, 122))),
            pl.BlockSpec(
                (D, BLKW // 2), lambda j: (0, jnp.minimum(2 * j + 1, 122))),
        ],
        out_specs=pl.BlockSpec((BLKW // 2, 128), lambda j: (j, 0)),
        out_shape=jax.ShapeDtypeStruct((VPAD // 2, 128), jnp.float32),
    )(tT, tT)


def _tc_transpose_lora(aT):
    def body(x_ref, o_ref):
        x = x_ref[...]
        w = BLKW // 8
        stacked = jnp.concatenate(
            [lax.slice(x, (0, k * w), (R, (k + 1) * w)) for k in range(8)],
            axis=0)
        o_ref[...] = _mxu_t(stacked)

    return pl.pallas_call(
        body,
        grid=(NBLK,),
        in_specs=[pl.BlockSpec((R, BLKW), lambda j: (0, j))],
        out_specs=pl.BlockSpec((BLKW // 8, 128), lambda j: (j, 0)),
        out_shape=jax.ShapeDtypeStruct((VPAD // 8, 128), jnp.float32),
    )(aT)


@functools.partial(jax.jit, static_argnums=(3, 4))
def _sc_gather(idx3, t64, a16, n_tokens, n_rows):
    per_w = n_rows * 128
    n_chunks = per_w // CT
    mesh = plsc.VectorSubcoreMesh(core_axis_name="c", subcore_axis_name="s")

    @functools.partial(
        pl.kernel,
        mesh=mesh,
        out_type=(
            jax.ShapeDtypeStruct((n_tokens, D), jnp.float32),
            jax.ShapeDtypeStruct((n_tokens // 2, 128), jnp.float32),
        ),
        scratch_types=[
            pltpu.VMEM((n_rows, 128), jnp.int32),
            pltpu.VMEM((n_rows, 128), jnp.int32),
            pltpu.VMEM((n_rows, 128), jnp.int32),
            pltpu.VMEM((NSLOT, CT, D), jnp.float32),
            pltpu.VMEM((NSLOT, CT, R), jnp.float32),
            pltpu.VMEM((NSLOT, CT // 2, 128), jnp.float32),
            pltpu.SemaphoreType.DMA,
            pltpu.SemaphoreType.DMA,
            pltpu.SemaphoreType.DMA,
            pltpu.SemaphoreType.DMA,
        ],
        compiler_params=pltpu.CompilerParams(use_tc_tiling_on_sc=False),
    )
    def k(idx_hbm, t_hbm, a_hbm, base_out, hid_out, idx_v, mt_v, ml_v,
          rows_v, hv_v, pv_v, g0, g1, w0, w1):
        gsem = (g0, g1)
        wsem = (w0, w1)
        cid = lax.axis_index("c")
        sid = lax.axis_index("s")
        wid = sid * NC + cid
        pltpu.sync_copy(idx_hbm.at[wid], idx_v)

        def xform(r, _):
            for g in range(8):
                v = idx_v[r, pl.ds(g * 16, 16)]
                hi = (v >> 14) << 14
                mt = hi + ((v & 8191) << 1) + ((v >> 13) & 1)
                ml = hi + ((v & 2047) << 3) + ((v >> 11) & 7)
                mt_v[r, pl.ds(g * 16, 16)] = mt
                ml_v[r, pl.ds(g * 16, 16)] = ml
            return 0

        lax.fori_loop(0, n_rows, xform, 0)

        zero = jnp.zeros((16,), jnp.float32)

        def zpad(r, _):
            for s in range(NSLOT):
                for g in range(2, 8):
                    pv_v[s, r, pl.ds(g * 16, 16)] = zero
            return 0

        lax.fori_loop(0, CT // 2, zpad, 0)

        def fire_gather(kk, s):
            pltpu.make_async_copy(
                t_hbm.at[mt_v.at[kk]], rows_v.at[s], gsem[s]).start()
            pltpu.make_async_copy(
                a_hbm.at[ml_v.at[kk]], hv_v.at[s], gsem[s]).start()

        def wait_gather(kk, s):
            pltpu.make_async_copy(
                t_hbm.at[mt_v.at[kk]], rows_v.at[s], gsem[s]).wait()
            pltpu.make_async_copy(
                a_hbm.at[ml_v.at[kk]], hv_v.at[s], gsem[s]).wait()

        def pack_pairs(s):
            for r in range(CT // 2):
                pv_v[s, r, pl.ds(0, 16)] = hv_v[s, 2 * r, :]
                pv_v[s, r, pl.ds(16, 16)] = hv_v[s, 2 * r + 1, :]

        def _base_slice(kk):
            return pl.ds(pl.multiple_of(wid * per_w + kk * CT, CT), CT)

        def _hid_slice(kk):
            return pl.ds(
                pl.multiple_of((wid * per_w + kk * CT) // 2, CT // 2), CT // 2)

        def fire_write(kk, s):
            pltpu.make_async_copy(
                rows_v.at[s], base_out.at[_base_slice(kk)], wsem[s]).start()
            pltpu.make_async_copy(
                pv_v.at[s], hid_out.at[_hid_slice(kk)], wsem[s]).start()

        def wait_write(kk, s):
            pltpu.make_async_copy(
                rows_v.at[s], base_out.at[_base_slice(kk)], wsem[s]).wait()
            pltpu.make_async_copy(
                pv_v.at[s], hid_out.at[_hid_slice(kk)], wsem[s]).wait()

        def step(kk, s, fire_next):
            wait_gather(kk, s)
            pack_pairs(s)
            fire_write(kk, s)
            wait_write(kk, s)
            if fire_next:
                fire_gather(kk + NSLOT, s)

        for s in range(NSLOT):
            fire_gather(s, s)

        def chunk(i, _):
            for s in range(NSLOT):
                step(NSLOT * i + s, s, True)
            return 0

        lax.fori_loop(0, n_chunks // NSLOT - 1, chunk, 0)
        for s in range(NSLOT):
            step(n_chunks - NSLOT + s, s, False)

    return k(idx3, t64, a16)


def _build_w(lora_b):
    w = jnp.zeros((128, 128), jnp.float32)
    w = w.at[0:R, 0:D].set(lora_b)
    w = w.at[R:2 * R, D:2 * D].set(lora_b)
    return w


def _tc_combine(base2, hid2, w, n2):
    blk = 4096

    def body(b_ref, h_ref, w_ref, o_ref):
        o_ref[...] = b_ref[...] + jnp.dot(
            h_ref[...], w_ref[...], preferred_element_type=jnp.float32)

    return pl.pallas_call(
        body,
        grid=(n2 // blk,),
        in_specs=[
            pl.BlockSpec((blk, 128), lambda i: (i, 0)),
            pl.BlockSpec((blk, 128), lambda i: (i, 0)),
            pl.BlockSpec((128, 128), lambda i: (0, 0)),
        ],
        out_specs=pl.BlockSpec((blk, 128), lambda i: (i, 0)),
        out_shape=jax.ShapeDtypeStruct((n2, 128), jnp.float32),
    )(base2, hid2, w)


def kernel(x, table, lora_A, lora_B):
    b, s = x.shape
    n = b * s
    n_rows = n // (NW * 128)
    tconv = _tc_transpose_table(table.T)
    aconv = _tc_transpose_lora(lora_A.T)
    t64 = tconv.reshape(VPAD, D)
    a16 = aconv.reshape(VPAD, R)
    idx3 = x.reshape(NW, n_rows, 128).astype(jnp.int32)
    base, hid2 = _sc_gather(idx3, t64, a16, n, n_rows)
    base2 = base.reshape(n // 2, 128)
    out2 = _tc_combine(base2, hid2, _build_w(lora_B), n // 2)
    return out2.reshape(b, s, D)
